# baseline (speedup 1.0000x reference)
import functools

import jax
import jax.numpy as jnp
from jax import lax
from jax.experimental import pallas as pl
from jax.experimental.pallas import tpu as pltpu
from jax.experimental.pallas import tpu_sc as plsc

N_EMB = 1000000
D_EMB = 64
BATCH = 16384

_info = plsc.get_sparse_core_info()
_NC, _NS = _info.num_cores, _info.num_subcores
_NW = _NC * _NS
_SHARD = 31232
_CHUNK = 512
_NCHUNK = _SHARD // _CHUNK
_DUMP = BATCH
_SCR_ROWS = BATCH + 8

_mesh = plsc.VectorSubcoreMesh(core_axis_name="c", subcore_axis_name="s")
_params = pltpu.CompilerParams(needs_layout_passes=False)


@functools.partial(
    pl.kernel,
    mesh=_mesh,
    out_type=jax.ShapeDtypeStruct((_SCR_ROWS, 128), jnp.float32),
    compiler_params=_params,
    scratch_types=[
        pltpu.VMEM((BATCH,), jnp.int32),
        pltpu.VMEM((BATCH,), jnp.int32),
        pltpu.VMEM((BATCH,), jnp.int32),
        pltpu.VMEM((BATCH,), jnp.int32),
        pltpu.VMEM((BATCH,), jnp.int32),
        pltpu.VMEM((D_EMB, _CHUNK), jnp.float32),
        pltpu.VMEM((16, 128), jnp.float32),
        pltpu.VMEM((1, 16), jnp.int32),
        pltpu.SemaphoreType.DMA,
    ],
)
def _scan_gather(x_hbm, embt_hbm, tail_hbm, out_hbm, idx_v, mb_v, mr_v,
                 cb_v, cr_v, stage_v, rows_v, sidx_v, sem):
    wid = lax.axis_index("s") * _NC + lax.axis_index("c")
    lo = wid * _SHARD
    hi = jnp.where(wid == _NW - 1, N_EMB, lo + _SHARD).astype(jnp.int32)
    iota = lax.iota(jnp.int32, 16)

    pltpu.sync_copy(x_hbm, idx_v)

    def prefilter(g, off):
        v = idx_v[pl.ds(g * 16, 16)]
        m = (v >= lo) & (v < hi)
        plsc.store_compressed(mb_v.at[pl.ds(off, 16)], g * 16 + iota, mask=m)
        plsc.store_compressed(mr_v.at[pl.ds(off, 16)], v, mask=m)
        return off + plsc.all_reduce_population_count(m)[0]

    n_match = lax.fori_loop(0, BATCH // 16, prefilter, jnp.int32(0))
    n_groups = (n_match + 15) // 16

    def process_chunk(rlo, width):
        rhi = rlo + width

        def rescan(g, off):
            rv = mr_v[pl.ds(g * 16, 16)]
            bv = mb_v[pl.ds(g * 16, 16)]
            m = (rv >= rlo) & (rv < rhi)
            plsc.store_compressed(cb_v.at[pl.ds(off, 16)], bv, mask=m)
            plsc.store_compressed(cr_v.at[pl.ds(off, 16)], rv, mask=m)
            return off + plsc.all_reduce_population_count(m)[0]

        n2 = lax.fori_loop(0, n_groups, rescan, jnp.int32(0))

        def extract(h, _):
            b16 = cb_v[pl.ds(h * 16, 16)]
            r16 = cr_v[pl.ds(h * 16, 16)] - rlo
            mk = (h * 16 + iota) < n2
            r16 = jnp.where(mk, r16, 0)
            for dd in range(D_EMB):
                dsplat = jnp.full((16,), dd, jnp.int32)
                val = plsc.load_gather(stage_v, [dsplat, r16], mask=mk)
                plsc.store_scatter(rows_v, [iota, dsplat], val, mask=mk)
            sidx_v[0, :] = jnp.where(mk, b16, _DUMP)
            cp = pltpu.async_copy(rows_v, out_hbm.at[sidx_v.at[0]], sem)
            cp.wait()
            return 0

        lax.fori_loop(0, (n2 + 15) // 16, extract, 0)

    def chunk_body(c, _):
        rlo = lo + c * _CHUNK
        pltpu.sync_copy(
            embt_hbm.at[:, pl.ds(pl.multiple_of(rlo, 128), _CHUNK)],
            stage_v.at[:, pl.ds(0, _CHUNK)])
        process_chunk(rlo, _CHUNK)
        return 0

    lax.fori_loop(0, _NCHUNK, chunk_body, 0)

    @pl.when(wid == _NW - 1)
    def _tail():
        rlo = jnp.int32(_NW * _SHARD)
        pltpu.sync_copy(
            embt_hbm.at[:, pl.ds(pl.multiple_of(rlo, 128), _CHUNK)],
            stage_v.at[:, pl.ds(0, _CHUNK)])
        process_chunk(rlo, _CHUNK)
        pltpu.sync_copy(tail_hbm, stage_v.at[:, pl.ds(0, 128)])
        process_chunk(jnp.int32(_NW * _SHARD + _CHUNK), 128)


_BPW = BATCH // _NW


@functools.partial(
    pl.kernel,
    mesh=_mesh,
    out_type=jax.ShapeDtypeStruct((D_EMB, BATCH), jnp.float32),
    compiler_params=_params,
    scratch_types=[
        pltpu.VMEM((_BPW, 128), jnp.float32),
        pltpu.VMEM((D_EMB, _BPW), jnp.float32),
        pltpu.SemaphoreType.DMA,
    ],
)
def _transpose_out(scr_hbm, out_hbm, st_v, ob_v, sem):
    wid = lax.axis_index("s") * _NC + lax.axis_index("c")
    b0 = wid * _BPW
    iota = lax.iota(jnp.int32, 16)
    pltpu.sync_copy(scr_hbm.at[pl.ds(pl.multiple_of(b0, 8), _BPW)], st_v)

    def grp(h, _):
        b16 = h * 16 + iota
        for dd in range(D_EMB):
            val = plsc.load_gather(st_v, [b16, jnp.full((16,), dd, jnp.int32)])
            ob_v[dd, pl.ds(h * 16, 16)] = val
        return 0

    lax.fori_loop(0, _BPW // 16, grp, 0)
    pltpu.sync_copy(ob_v, out_hbm.at[:, pl.ds(pl.multiple_of(b0, 128), _BPW)])


def kernel(x, emb):
    tail = jnp.zeros((D_EMB, 128), jnp.float32)
    tail = tail.at[:, : N_EMB - _NW * _SHARD - _CHUNK].set(
        emb[_NW * _SHARD + _CHUNK:].T)
    scr = _scan_gather(x.astype(jnp.int32), emb.T, tail)
    out_t = _transpose_out(scr)
    return out_t.T


# speedup vs baseline: 1.0024x; 1.0024x over previous
import functools

import jax
import jax.numpy as jnp
from jax import lax
from jax.experimental import pallas as pl
from jax.experimental.pallas import tpu as pltpu
from jax.experimental.pallas import tpu_sc as plsc

N_EMB = 1000000
D_EMB = 64
BATCH = 16384

_info = plsc.get_sparse_core_info()
_NC, _NS = _info.num_cores, _info.num_subcores
_NW = _NC * _NS
_SHARD = 31232
_CHUNK = 512
_NCHUNK = _SHARD // _CHUNK
_DUMP = BATCH
_SCR_ROWS = BATCH + 8

_mesh = plsc.VectorSubcoreMesh(core_axis_name="c", subcore_axis_name="s")
_params = pltpu.CompilerParams(needs_layout_passes=False)


@functools.partial(
    pl.kernel,
    mesh=_mesh,
    out_type=jax.ShapeDtypeStruct((_SCR_ROWS, 128), jnp.float32),
    compiler_params=_params,
    scratch_types=[
        pltpu.VMEM((BATCH,), jnp.int32),
        pltpu.VMEM((BATCH,), jnp.int32),
        pltpu.VMEM((BATCH,), jnp.int32),
        pltpu.VMEM((BATCH,), jnp.int32),
        pltpu.VMEM((BATCH,), jnp.int32),
        pltpu.VMEM((D_EMB, _CHUNK), jnp.float32),
        pltpu.VMEM((16, 128), jnp.float32),
        pltpu.VMEM((1, 16), jnp.int32),
        pltpu.SemaphoreType.DMA,
    ],
)
def _scan_gather(x_hbm, embt_hbm, tail_hbm, out_hbm, idx_v, mb_v, mr_v,
                 cb_v, cr_v, stage_v, rows_v, sidx_v, sem):
    wid = lax.axis_index("s") * _NC + lax.axis_index("c")
    lo = wid * _SHARD
    hi = jnp.where(wid == _NW - 1, N_EMB, lo + _SHARD).astype(jnp.int32)
    iota = lax.iota(jnp.int32, 16)

    pltpu.sync_copy(x_hbm, idx_v)

    def prefilter(g, off):
        v = idx_v[pl.ds(g * 16, 16)]
        m = (v >= lo) & (v < hi)
        plsc.store_compressed(mb_v.at[pl.ds(off, 16)], g * 16 + iota, mask=m)
        plsc.store_compressed(mr_v.at[pl.ds(off, 16)], v, mask=m)
        return off + plsc.all_reduce_population_count(m)[0]

    n_match = lax.fori_loop(0, BATCH // 16, prefilter, jnp.int32(0))
    n_groups = (n_match + 15) // 16

    def process_chunk(rlo, width):
        rhi = rlo + width

        def rescan(g, off):
            rv = mr_v[pl.ds(g * 16, 16)]
            bv = mb_v[pl.ds(g * 16, 16)]
            m = (rv >= rlo) & (rv < rhi)
            plsc.store_compressed(cb_v.at[pl.ds(off, 16)], bv, mask=m)
            plsc.store_compressed(cr_v.at[pl.ds(off, 16)], rv, mask=m)
            return off + plsc.all_reduce_population_count(m)[0]

        n2 = lax.fori_loop(0, n_groups, rescan, jnp.int32(0))

        def extract(h, _):
            b16 = cb_v[pl.ds(h * 16, 16)]
            r16 = cr_v[pl.ds(h * 16, 16)] - rlo
            mk = (h * 16 + iota) < n2
            r16 = jnp.where(mk, r16, 0)
            for dd in range(D_EMB):
                dsplat = jnp.full((16,), dd, jnp.int32)
                val = plsc.load_gather(stage_v, [dsplat, r16], mask=mk)
                plsc.store_scatter(rows_v, [iota, dsplat], val, mask=mk)
            sidx_v[0, :] = jnp.where(mk, b16, _DUMP)
            cp = pltpu.async_copy(rows_v, out_hbm.at[sidx_v.at[0]], sem)
            cp.wait()
            return 0

        lax.fori_loop(0, (n2 + 15) // 16, extract, 0)

    def stage_chunk(rlo):
        cps = [
            pltpu.async_copy(
                embt_hbm.at[pl.ds(i * 8, 8),
                            pl.ds(pl.multiple_of(rlo, 128), _CHUNK)],
                stage_v.at[pl.ds(i * 8, 8), pl.ds(0, _CHUNK)],
                sem)
            for i in range(8)
        ]
        for cp in cps:
            cp.wait()

    def chunk_body(c, _):
        rlo = lo + c * _CHUNK
        stage_chunk(rlo)
        process_chunk(rlo, _CHUNK)
        return 0

    lax.fori_loop(0, _NCHUNK, chunk_body, 0)

    @pl.when(wid == _NW - 1)
    def _tail():
        rlo = jnp.int32(_NW * _SHARD)
        stage_chunk(rlo)
        process_chunk(rlo, _CHUNK)
        pltpu.sync_copy(tail_hbm, stage_v.at[:, pl.ds(0, 128)])
        process_chunk(jnp.int32(_NW * _SHARD + _CHUNK), 128)


_BPW = BATCH // _NW


@functools.partial(
    pl.kernel,
    mesh=_mesh,
    out_type=jax.ShapeDtypeStruct((D_EMB, BATCH), jnp.float32),
    compiler_params=_params,
    scratch_types=[
        pltpu.VMEM((_BPW, 128), jnp.float32),
        pltpu.VMEM((D_EMB, _BPW), jnp.float32),
        pltpu.SemaphoreType.DMA,
    ],
)
def _transpose_out(scr_hbm, out_hbm, st_v, ob_v, sem):
    wid = lax.axis_index("s") * _NC + lax.axis_index("c")
    b0 = wid * _BPW
    iota = lax.iota(jnp.int32, 16)
    pltpu.sync_copy(scr_hbm.at[pl.ds(pl.multiple_of(b0, 8), _BPW)], st_v)

    def grp(h, _):
        b16 = h * 16 + iota
        for dd in range(D_EMB):
            val = plsc.load_gather(st_v, [b16, jnp.full((16,), dd, jnp.int32)])
            ob_v[dd, pl.ds(h * 16, 16)] = val
        return 0

    lax.fori_loop(0, _BPW // 16, grp, 0)
    pltpu.sync_copy(ob_v, out_hbm.at[:, pl.ds(pl.multiple_of(b0, 128), _BPW)])


def kernel(x, emb):
    tail = jnp.zeros((D_EMB, 128), jnp.float32)
    tail = tail.at[:, : N_EMB - _NW * _SHARD - _CHUNK].set(
        emb[_NW * _SHARD + _CHUNK:].T)
    scr = _scan_gather(x.astype(jnp.int32), emb.T, tail)
    out_t = _transpose_out(scr)
    return out_t.T
